# Initial kernel scaffold; baseline (speedup 1.0000x reference)
import jax
import jax.numpy as jnp
from jax.experimental import pallas as pl


def kernel(x, weight):
    raise NotImplementedError("write your pallas kernel here")


# speedup vs baseline: 1.0623x; 1.0623x over previous
import functools

import jax
import jax.numpy as jnp
from jax import lax
from jax.experimental import pallas as pl
from jax.experimental.pallas import tpu as pltpu
from jax.experimental.pallas import tpu_sc as plsc

_COMMITMENT_COST = 0.25

_BM = 256
_BW = 4096
_BL = 2048


def _l2_normalize(v, axis, eps=1e-12):
    norm = jnp.sqrt(jnp.sum(v * v, axis=axis, keepdims=True))
    return v / jnp.maximum(norm, eps)


def _argmin_body(xn_ref, wn_ref, sx_ref, sw_ref, idx_ref):
    n = wn_ref.shape[0]
    xb = xn_ref[...].astype(jnp.bfloat16)
    wb = wn_ref[...].astype(jnp.bfloat16)
    dot = lax.dot_general(xb, wb, (((1,), (1,)), ((), ())),
                          preferred_element_type=jnp.float32)
    dist = sx_ref[...] + sw_ref[...] - 2.0 * dot
    cols = lax.broadcasted_iota(jnp.int32, (_BM, n), 1)
    big = jnp.iinfo(jnp.int32).max
    run_i = None
    run_v = None
    for s in range(0, n, _BW):
        m = (cols >= s) & (cols < min(s + _BW, n))
        wdist = jnp.where(m, dist, jnp.inf)
        lv = jnp.min(wdist, axis=1, keepdims=True)
        li = jnp.min(jnp.where(wdist == lv, cols, big), axis=1, keepdims=True)
        if run_i is None:
            run_i, run_v = li, lv
        else:
            upd = lv < run_v
            run_i = jnp.where(upd, li, run_i)
            run_v = jnp.where(upd, lv, run_v)
        run_v = run_v.astype(jnp.bfloat16).astype(jnp.float32)
    idx_ref[...] = run_i


def _argmin_call(xn, wn, sx, sw):
    m, k = xn.shape
    n = wn.shape[0]
    return pl.pallas_call(
        _argmin_body,
        grid=(m // _BM,),
        in_specs=[
            pl.BlockSpec((_BM, k), lambda i: (i, 0)),
            pl.BlockSpec((n, k), lambda i: (0, 0)),
            pl.BlockSpec((_BM, 1), lambda i: (i, 0)),
            pl.BlockSpec((1, n), lambda i: (0, 0)),
        ],
        out_specs=pl.BlockSpec((_BM, 1), lambda i: (i, 0)),
        out_shape=jax.ShapeDtypeStruct((m, 1), jnp.int32),
        compiler_params=pltpu.CompilerParams(
            dimension_semantics=("arbitrary",)),
    )(xn, wn, sx, sw)


@functools.lru_cache(maxsize=None)
def _make_gather(v, d, b):
    info = plsc.get_sparse_core_info()
    nc, ns = info.num_cores, info.num_subcores
    nw = nc * ns
    ch = 128
    b_per_w = b // nw
    n_ch = b_per_w // ch
    mesh = plsc.VectorSubcoreMesh(core_axis_name="c", subcore_axis_name="s")

    @functools.partial(
        pl.kernel, mesh=mesh,
        out_type=jax.ShapeDtypeStruct((b, d), jnp.float32),
        scratch_types=[
            pltpu.VMEM((ch,), jnp.int32),
            pltpu.VMEM((ch, d), jnp.float32),
            pltpu.SemaphoreType.DMA,
        ],
    )
    def gath(table_hbm, idx_hbm, out_hbm, idx_v, rows_v, sem):
        wid = lax.axis_index("s") * nc + lax.axis_index("c")
        for c in range(n_ch):
            base = wid * b_per_w + c * ch
            pltpu.sync_copy(idx_hbm.at[pl.ds(base, ch)], idx_v)
            pltpu.async_copy(table_hbm.at[idx_v], rows_v, sem).wait()
            pltpu.sync_copy(rows_v, out_hbm.at[pl.ds(base, ch)])

    return gath


def _loss_body(x_ref, q_ref, acc_ref):
    i = pl.program_id(0)
    d = q_ref[...] - x_ref[...]
    s = jnp.sum(d * d)

    @pl.when(i == 0)
    def _init():
        acc_ref[0, 0] = s

    @pl.when(i > 0)
    def _acc():
        acc_ref[0, 0] += s


def _loss_call(x, q):
    m, k = x.shape
    return pl.pallas_call(
        _loss_body,
        grid=(m // _BL,),
        in_specs=[
            pl.BlockSpec((_BL, k), lambda i: (i, 0)),
            pl.BlockSpec((_BL, k), lambda i: (i, 0)),
        ],
        out_specs=pl.BlockSpec((1, 1), lambda i: (0, 0),
                               memory_space=pltpu.SMEM),
        out_shape=jax.ShapeDtypeStruct((1, 1), jnp.float32),
        compiler_params=pltpu.CompilerParams(
            dimension_semantics=("arbitrary",)),
    )(x, q)


def kernel(x, weight):
    m, k = x.shape
    n = weight.shape[0]
    flat_x = _l2_normalize(x, axis=1)
    w_norm = _l2_normalize(weight, axis=1)
    sx = jnp.sum(flat_x ** 2, axis=1, keepdims=True)
    sw = jnp.sum(w_norm ** 2, axis=1)[None, :]

    idx = _argmin_call(flat_x, w_norm, sx, sw).reshape(-1)
    q = _make_gather(n, k, m)(weight, idx)
    ssq = _loss_call(x, q)
    mean_sq = ssq[0, 0] / (m * k)
    loss = mean_sq + _COMMITMENT_COST * mean_sq
    return q, loss, idx


# speedup vs baseline: 1.3717x; 1.2912x over previous
import functools

import jax
import jax.numpy as jnp
from jax import lax
from jax.experimental import pallas as pl
from jax.experimental.pallas import tpu as pltpu
from jax.experimental.pallas import tpu_sc as plsc

_COMMITMENT_COST = 0.25

_BM = 256
_BW = 4096
_BL = 2048


def _l2_normalize(v, axis, eps=1e-12):
    norm = jnp.sqrt(jnp.sum(v * v, axis=axis, keepdims=True))
    return v / jnp.maximum(norm, eps)


def _argmin_body(xb_ref, wb_ref, sx_ref, sw_ref, idx_ref):
    n = wb_ref.shape[0]
    dot = lax.dot_general(xb_ref[...], wb_ref[...], (((1,), (1,)), ((), ())),
                          preferred_element_type=jnp.float32)
    sx = sx_ref[...]
    iota = lax.broadcasted_iota(jnp.int32, (_BM, 128), 1).astype(jnp.float32)
    inf = jnp.float32(jnp.inf)
    run_i = None
    run_v = None
    for base in range(0, n, _BW):
        acc_v = acc_c = None
        for s in range(base, base + _BW, 128):
            d = (sx + sw_ref[:, s:s + 128]) - 2.0 * dot[:, s:s + 128]
            c = iota + jnp.float32(s)
            if acc_v is None:
                acc_v, acc_c = d, c
            else:
                m = d < acc_v
                acc_v = jnp.where(m, d, acc_v)
                acc_c = jnp.where(m, c, acc_c)
        lv = jnp.min(acc_v, axis=1, keepdims=True)
        li = jnp.min(jnp.where(acc_v == lv, acc_c, inf), axis=1,
                     keepdims=True)
        if run_i is None:
            run_i, run_v = li, lv
        else:
            upd = lv < run_v
            run_i = jnp.where(upd, li, run_i)
            run_v = jnp.where(upd, lv, run_v)
        run_v = run_v.astype(jnp.bfloat16).astype(jnp.float32)
    idx_ref[...] = run_i.astype(jnp.int32)


def _argmin_call(xb, wb, sx, sw):
    m, k = xb.shape
    n = wb.shape[0]
    return pl.pallas_call(
        _argmin_body,
        grid=(m // _BM,),
        in_specs=[
            pl.BlockSpec((_BM, k), lambda i: (i, 0)),
            pl.BlockSpec((n, k), lambda i: (0, 0)),
            pl.BlockSpec((_BM, 1), lambda i: (i, 0)),
            pl.BlockSpec((1, n), lambda i: (0, 0)),
        ],
        out_specs=pl.BlockSpec((_BM, 1), lambda i: (i, 0)),
        out_shape=jax.ShapeDtypeStruct((m, 1), jnp.int32),
        compiler_params=pltpu.CompilerParams(
            dimension_semantics=("arbitrary",)),
    )(xb, wb, sx, sw)


@functools.lru_cache(maxsize=None)
def _make_gather(v, d, b):
    info = plsc.get_sparse_core_info()
    nc, ns = info.num_cores, info.num_subcores
    nw = nc * ns
    ch = 128
    b_per_w = b // nw
    n_ch = b_per_w // ch
    mesh = plsc.VectorSubcoreMesh(core_axis_name="c", subcore_axis_name="s")

    @functools.partial(
        pl.kernel, mesh=mesh,
        out_type=jax.ShapeDtypeStruct((b, d), jnp.float32),
        scratch_types=[
            pltpu.VMEM((ch,), jnp.int32),
            pltpu.VMEM((ch, d), jnp.float32),
            pltpu.SemaphoreType.DMA,
        ],
    )
    def gath(table_hbm, idx_hbm, out_hbm, idx_v, rows_v, sem):
        wid = lax.axis_index("s") * nc + lax.axis_index("c")
        for c in range(n_ch):
            base = wid * b_per_w + c * ch
            pltpu.sync_copy(idx_hbm.at[pl.ds(base, ch)], idx_v)
            pltpu.async_copy(table_hbm.at[idx_v], rows_v, sem).wait()
            pltpu.sync_copy(rows_v, out_hbm.at[pl.ds(base, ch)])

    return gath


def _loss_body(x_ref, q_ref, acc_ref):
    i = pl.program_id(0)
    d = q_ref[...] - x_ref[...]
    s = jnp.sum(d * d)

    @pl.when(i == 0)
    def _init():
        acc_ref[0, 0] = s

    @pl.when(i > 0)
    def _acc():
        acc_ref[0, 0] += s


def _loss_call(x, q):
    m, k = x.shape
    return pl.pallas_call(
        _loss_body,
        grid=(m // _BL,),
        in_specs=[
            pl.BlockSpec((_BL, k), lambda i: (i, 0)),
            pl.BlockSpec((_BL, k), lambda i: (i, 0)),
        ],
        out_specs=pl.BlockSpec((1, 1), lambda i: (0, 0),
                               memory_space=pltpu.SMEM),
        out_shape=jax.ShapeDtypeStruct((1, 1), jnp.float32),
        compiler_params=pltpu.CompilerParams(
            dimension_semantics=("arbitrary",)),
    )(x, q)


def kernel(x, weight):
    m, k = x.shape
    n = weight.shape[0]
    flat_x = _l2_normalize(x, axis=1)
    w_norm = _l2_normalize(weight, axis=1)
    sx = jnp.sum(flat_x ** 2, axis=1, keepdims=True)
    sw = jnp.sum(w_norm ** 2, axis=1)[None, :]

    idx = _argmin_call(flat_x.astype(jnp.bfloat16),
                       w_norm.astype(jnp.bfloat16), sx, sw).reshape(-1)
    q = _make_gather(n, k, m)(weight, idx)
    ssq = _loss_call(x, q)
    mean_sq = ssq[0, 0] / (m * k)
    loss = mean_sq + _COMMITMENT_COST * mean_sq
    return q, loss, idx


# speedup vs baseline: 1.3928x; 1.0154x over previous
import functools

import jax
import jax.numpy as jnp
from jax import lax
from jax.experimental import pallas as pl
from jax.experimental.pallas import tpu as pltpu
from jax.experimental.pallas import tpu_sc as plsc

_COMMITMENT_COST = 0.25

_BM = 256
_BW = 4096
_BL = 2048


def _l2_normalize(v, axis, eps=1e-12):
    norm = jnp.sqrt(jnp.sum(v * v, axis=axis, keepdims=True))
    return v / jnp.maximum(norm, eps)


def _argmin_body(xb_ref, wb_ref, sx_ref, sw_ref, idx_ref):
    n = wb_ref.shape[0]
    dot = lax.dot_general(xb_ref[...], wb_ref[...], (((1,), (1,)), ((), ())),
                          preferred_element_type=jnp.float32)
    sx = sx_ref[...]
    iota = lax.broadcasted_iota(jnp.int32, (_BM, 128), 1).astype(jnp.float32)
    inf = jnp.float32(jnp.inf)
    run_i = None
    run_v = None
    for base in range(0, n, _BW):
        acc_v = acc_c = None
        for s in range(base, base + _BW, 128):
            d = (sx + sw_ref[:, s:s + 128]) - 2.0 * dot[:, s:s + 128]
            c = iota + jnp.float32(s)
            if acc_v is None:
                acc_v, acc_c = d, c
            else:
                m = d < acc_v
                acc_v = jnp.where(m, d, acc_v)
                acc_c = jnp.where(m, c, acc_c)
        lv = jnp.min(acc_v, axis=1, keepdims=True)
        li = jnp.min(jnp.where(acc_v == lv, acc_c, inf), axis=1,
                     keepdims=True)
        if run_i is None:
            run_i, run_v = li, lv
        else:
            upd = lv < run_v
            run_i = jnp.where(upd, li, run_i)
            run_v = jnp.where(upd, lv, run_v)
        run_v = run_v.astype(jnp.bfloat16).astype(jnp.float32)
    idx_ref[...] = run_i.astype(jnp.int32)


def _argmin_call(xb, wb, sx, sw):
    m, k = xb.shape
    n = wb.shape[0]
    return pl.pallas_call(
        _argmin_body,
        grid=(m // _BM,),
        in_specs=[
            pl.BlockSpec((_BM, k), lambda i: (i, 0)),
            pl.BlockSpec((n, k), lambda i: (0, 0)),
            pl.BlockSpec((_BM, 1), lambda i: (i, 0)),
            pl.BlockSpec((1, n), lambda i: (0, 0)),
        ],
        out_specs=pl.BlockSpec((_BM, 1), lambda i: (i, 0)),
        out_shape=jax.ShapeDtypeStruct((m, 1), jnp.int32),
        compiler_params=pltpu.CompilerParams(
            dimension_semantics=("arbitrary",)),
    )(xb, wb, sx, sw)


@functools.lru_cache(maxsize=None)
def _make_gather(v, d, b):
    info = plsc.get_sparse_core_info()
    nc, ns = info.num_cores, info.num_subcores
    nw = nc * ns
    ch = 128
    b_per_w = b // nw
    n_ch = b_per_w // ch
    mesh = plsc.VectorSubcoreMesh(core_axis_name="c", subcore_axis_name="s")

    @functools.partial(
        pl.kernel, mesh=mesh,
        out_type=jax.ShapeDtypeStruct((b, d), jnp.float32),
        scratch_types=[
            pltpu.VMEM((ch,), jnp.int32),
            pltpu.VMEM((ch, d), jnp.float32),
            pltpu.SemaphoreType.DMA,
        ],
    )
    def gath(table_hbm, idx_hbm, out_hbm, idx_v, rows_v, sem):
        wid = lax.axis_index("s") * nc + lax.axis_index("c")
        for c in range(n_ch):
            base = wid * b_per_w + c * ch
            pltpu.sync_copy(idx_hbm.at[pl.ds(base, ch)], idx_v)
            pltpu.async_copy(table_hbm.at[idx_v], rows_v, sem).wait()
            pltpu.sync_copy(rows_v, out_hbm.at[pl.ds(base, ch)])

    return gath


def _loss_body(x_ref, q_ref, acc_ref):
    i = pl.program_id(0)
    d = q_ref[...] - x_ref[...]
    s = jnp.sum(d * d)

    @pl.when(i == 0)
    def _init():
        acc_ref[0, 0] = s

    @pl.when(i > 0)
    def _acc():
        acc_ref[0, 0] += s

    @pl.when(i == pl.num_programs(0) - 1)
    def _fin():
        total = x_ref.shape[0] * x_ref.shape[1] * pl.num_programs(0)
        mean_sq = acc_ref[0, 0] / total
        acc_ref[0, 0] = mean_sq + _COMMITMENT_COST * mean_sq


def _loss_call(x, q):
    m, k = x.shape
    return pl.pallas_call(
        _loss_body,
        grid=(m // _BL,),
        in_specs=[
            pl.BlockSpec((_BL, k), lambda i: (i, 0)),
            pl.BlockSpec((_BL, k), lambda i: (i, 0)),
        ],
        out_specs=pl.BlockSpec((1, 1), lambda i: (0, 0),
                               memory_space=pltpu.SMEM),
        out_shape=jax.ShapeDtypeStruct((1, 1), jnp.float32),
        compiler_params=pltpu.CompilerParams(
            dimension_semantics=("arbitrary",)),
    )(x, q)


def kernel(x, weight):
    m, k = x.shape
    n = weight.shape[0]
    flat_x = _l2_normalize(x, axis=1)
    w_norm = _l2_normalize(weight, axis=1)
    sx = jnp.sum(flat_x ** 2, axis=1, keepdims=True)
    sw = jnp.sum(w_norm ** 2, axis=1)[None, :]

    idx = _argmin_call(flat_x.astype(jnp.bfloat16),
                       w_norm.astype(jnp.bfloat16), sx, sw).reshape(-1)
    q = _make_gather(n, k, m)(weight, idx)
    loss = _loss_call(x, q)[0, 0]
    return q, loss, idx


# speedup vs baseline: 1.6169x; 1.1609x over previous
import functools

import jax
import jax.numpy as jnp
from jax import lax
from jax.experimental import pallas as pl
from jax.experimental.pallas import tpu as pltpu
from jax.experimental.pallas import tpu_sc as plsc

_COMMITMENT_COST = 0.25

_BM = 256
_BW = 4096
_BL = 2048


def _l2_normalize(v, axis, eps=1e-12):
    norm = jnp.sqrt(jnp.sum(v * v, axis=axis, keepdims=True))
    return v / jnp.maximum(norm, eps)


_RG = 32


def _argmin_body(xb_ref, wb_ref, sx_ref, sw_ref, idx_ref):
    n = wb_ref.shape[0]
    dotn = lax.dot_general(xb_ref[...], wb_ref[...], (((1,), (1,)), ((), ())),
                           preferred_element_type=jnp.float32)
    iota = lax.broadcasted_iota(jnp.int32, (_RG, 128), 1).astype(jnp.float32)
    inf = jnp.float32(jnp.inf)
    for g in range(_BM // _RG):
        r0 = g * _RG
        sx = sx_ref[r0:r0 + _RG, :]
        wins = []
        for base in range(0, n, _BW):
            acc_v = acc_c = None
            for s in range(base, base + _BW, 128):
                d = (sx + sw_ref[:, s:s + 128]) + dotn[r0:r0 + _RG,
                                                       s:s + 128]
                c = iota + jnp.float32(s)
                if acc_v is None:
                    acc_v, acc_c = d, c
                else:
                    m = d < acc_v
                    acc_v = jnp.where(m, d, acc_v)
                    acc_c = jnp.where(m, c, acc_c)
            lv = jnp.min(acc_v, axis=1, keepdims=True)
            li = jnp.min(jnp.where(acc_v == lv, acc_c, inf), axis=1,
                         keepdims=True)
            wins.append((lv, li))
        run_v, run_i = wins[0]
        run_v = run_v.astype(jnp.bfloat16).astype(jnp.float32)
        for lv, li in wins[1:]:
            upd = lv < run_v
            run_i = jnp.where(upd, li, run_i)
            run_v = jnp.where(upd, lv, run_v).astype(
                jnp.bfloat16).astype(jnp.float32)
        idx_ref[r0:r0 + _RG, :] = run_i.astype(jnp.int32)


def _argmin_call(xb, wb, sx, sw):
    m, k = xb.shape
    n = wb.shape[0]
    return pl.pallas_call(
        _argmin_body,
        grid=(m // _BM,),
        in_specs=[
            pl.BlockSpec((_BM, k), lambda i: (i, 0)),
            pl.BlockSpec((n, k), lambda i: (0, 0)),
            pl.BlockSpec((_BM, 1), lambda i: (i, 0)),
            pl.BlockSpec((1, n), lambda i: (0, 0)),
        ],
        out_specs=pl.BlockSpec((_BM, 1), lambda i: (i, 0)),
        out_shape=jax.ShapeDtypeStruct((m, 1), jnp.int32),
        compiler_params=pltpu.CompilerParams(
            dimension_semantics=("arbitrary",)),
    )(xb, wb, sx, sw)


@functools.lru_cache(maxsize=None)
def _make_gather(v, d, b):
    info = plsc.get_sparse_core_info()
    nc, ns = info.num_cores, info.num_subcores
    nw = nc * ns
    ch = 128
    b_per_w = b // nw
    n_ch = b_per_w // ch
    mesh = plsc.VectorSubcoreMesh(core_axis_name="c", subcore_axis_name="s")

    @functools.partial(
        pl.kernel, mesh=mesh,
        out_type=jax.ShapeDtypeStruct((b, d), jnp.float32),
        scratch_types=[
            pltpu.VMEM((ch,), jnp.int32),
            pltpu.VMEM((ch, d), jnp.float32),
            pltpu.SemaphoreType.DMA,
        ],
    )
    def gath(table_hbm, idx_hbm, out_hbm, idx_v, rows_v, sem):
        wid = lax.axis_index("s") * nc + lax.axis_index("c")
        for c in range(n_ch):
            base = wid * b_per_w + c * ch
            pltpu.sync_copy(idx_hbm.at[pl.ds(base, ch)], idx_v)
            pltpu.async_copy(table_hbm.at[idx_v], rows_v, sem).wait()
            pltpu.sync_copy(rows_v, out_hbm.at[pl.ds(base, ch)])

    return gath


def _loss_body(x_ref, q_ref, acc_ref):
    i = pl.program_id(0)
    d = q_ref[...] - x_ref[...]
    s = jnp.sum(d * d)

    @pl.when(i == 0)
    def _init():
        acc_ref[0, 0] = s

    @pl.when(i > 0)
    def _acc():
        acc_ref[0, 0] += s

    @pl.when(i == pl.num_programs(0) - 1)
    def _fin():
        total = x_ref.shape[0] * x_ref.shape[1] * pl.num_programs(0)
        mean_sq = acc_ref[0, 0] / total
        acc_ref[0, 0] = mean_sq + _COMMITMENT_COST * mean_sq


def _loss_call(x, q):
    m, k = x.shape
    return pl.pallas_call(
        _loss_body,
        grid=(m // _BL,),
        in_specs=[
            pl.BlockSpec((_BL, k), lambda i: (i, 0)),
            pl.BlockSpec((_BL, k), lambda i: (i, 0)),
        ],
        out_specs=pl.BlockSpec((1, 1), lambda i: (0, 0),
                               memory_space=pltpu.SMEM),
        out_shape=jax.ShapeDtypeStruct((1, 1), jnp.float32),
        compiler_params=pltpu.CompilerParams(
            dimension_semantics=("arbitrary",)),
    )(x, q)


def kernel(x, weight):
    m, k = x.shape
    n = weight.shape[0]
    flat_x = _l2_normalize(x, axis=1)
    w_norm = _l2_normalize(weight, axis=1)
    sx = jnp.sum(flat_x ** 2, axis=1, keepdims=True)
    sw = jnp.sum(w_norm ** 2, axis=1)[None, :]

    idx = _argmin_call((flat_x * jnp.float32(-2.0)).astype(jnp.bfloat16),
                       w_norm.astype(jnp.bfloat16), sx, sw).reshape(-1)
    q = _make_gather(n, k, m)(weight, idx)
    loss = _loss_call(x, q)[0, 0]
    return q, loss, idx


# speedup vs baseline: 1.6193x; 1.0015x over previous
import functools

import jax
import jax.numpy as jnp
from jax import lax
from jax.experimental import pallas as pl
from jax.experimental.pallas import tpu as pltpu
from jax.experimental.pallas import tpu_sc as plsc

_COMMITMENT_COST = 0.25

_BM = 256
_BW = 4096
_BL = 2048


def _l2_normalize(v, axis, eps=1e-12):
    norm = jnp.sqrt(jnp.sum(v * v, axis=axis, keepdims=True))
    return v / jnp.maximum(norm, eps)


_RG = 32


def _argmin_body(xn_ref, wn_ref, sx_ref, sw_ref, idx_ref, wb_ref):
    n = wn_ref.shape[0]

    @pl.when(pl.program_id(0) == 0)
    def _cast_w():
        wb_ref[...] = wn_ref[...].astype(jnp.bfloat16)

    xb = (xn_ref[...] * jnp.float32(-2.0)).astype(jnp.bfloat16)
    dotn = lax.dot_general(xb, wb_ref[...], (((1,), (1,)), ((), ())),
                           preferred_element_type=jnp.float32)
    iota = lax.broadcasted_iota(jnp.int32, (_RG, 128), 1).astype(jnp.float32)
    inf = jnp.float32(jnp.inf)
    for g in range(_BM // _RG):
        r0 = g * _RG
        sx = sx_ref[r0:r0 + _RG, :]
        wins = []
        for base in range(0, n, _BW):
            acc_v = acc_c = None
            for s in range(base, base + _BW, 128):
                d = (sx + sw_ref[:, s:s + 128]) + dotn[r0:r0 + _RG,
                                                       s:s + 128]
                c = iota + jnp.float32(s)
                if acc_v is None:
                    acc_v, acc_c = d, c
                else:
                    m = d < acc_v
                    acc_v = jnp.where(m, d, acc_v)
                    acc_c = jnp.where(m, c, acc_c)
            lv = jnp.min(acc_v, axis=1, keepdims=True)
            li = jnp.min(jnp.where(acc_v == lv, acc_c, inf), axis=1,
                         keepdims=True)
            wins.append((lv, li))
        run_v, run_i = wins[0]
        run_v = run_v.astype(jnp.bfloat16).astype(jnp.float32)
        for lv, li in wins[1:]:
            upd = lv < run_v
            run_i = jnp.where(upd, li, run_i)
            run_v = jnp.where(upd, lv, run_v).astype(
                jnp.bfloat16).astype(jnp.float32)
        idx_ref[r0:r0 + _RG, :] = run_i.astype(jnp.int32)


def _argmin_call(xn, wn, sx, sw):
    m, k = xn.shape
    n = wn.shape[0]
    return pl.pallas_call(
        _argmin_body,
        grid=(m // _BM,),
        in_specs=[
            pl.BlockSpec((_BM, k), lambda i: (i, 0)),
            pl.BlockSpec((n, k), lambda i: (0, 0)),
            pl.BlockSpec((_BM, 1), lambda i: (i, 0)),
            pl.BlockSpec((1, n), lambda i: (0, 0)),
        ],
        out_specs=pl.BlockSpec((_BM, 1), lambda i: (i, 0)),
        out_shape=jax.ShapeDtypeStruct((m, 1), jnp.int32),
        scratch_shapes=[pltpu.VMEM((n, k), jnp.bfloat16)],
        compiler_params=pltpu.CompilerParams(
            dimension_semantics=("arbitrary",)),
    )(xn, wn, sx, sw)


@functools.lru_cache(maxsize=None)
def _make_gather(v, d, b):
    info = plsc.get_sparse_core_info()
    nc, ns = info.num_cores, info.num_subcores
    nw = nc * ns
    ch = 128
    b_per_w = b // nw
    n_ch = b_per_w // ch
    mesh = plsc.VectorSubcoreMesh(core_axis_name="c", subcore_axis_name="s")

    @functools.partial(
        pl.kernel, mesh=mesh,
        out_type=jax.ShapeDtypeStruct((b, d), jnp.float32),
        scratch_types=[
            pltpu.VMEM((ch,), jnp.int32),
            pltpu.VMEM((ch, d), jnp.float32),
            pltpu.SemaphoreType.DMA,
        ],
    )
    def gath(table_hbm, idx_hbm, out_hbm, idx_v, rows_v, sem):
        wid = lax.axis_index("s") * nc + lax.axis_index("c")
        for c in range(n_ch):
            base = wid * b_per_w + c * ch
            pltpu.sync_copy(idx_hbm.at[pl.ds(base, ch)], idx_v)
            pltpu.async_copy(table_hbm.at[idx_v], rows_v, sem).wait()
            pltpu.sync_copy(rows_v, out_hbm.at[pl.ds(base, ch)])

    return gath


def _loss_body(x_ref, q_ref, acc_ref):
    i = pl.program_id(0)
    d = q_ref[...] - x_ref[...]
    s = jnp.sum(d * d)

    @pl.when(i == 0)
    def _init():
        acc_ref[0, 0] = s

    @pl.when(i > 0)
    def _acc():
        acc_ref[0, 0] += s

    @pl.when(i == pl.num_programs(0) - 1)
    def _fin():
        total = x_ref.shape[0] * x_ref.shape[1] * pl.num_programs(0)
        mean_sq = acc_ref[0, 0] / total
        acc_ref[0, 0] = mean_sq + _COMMITMENT_COST * mean_sq


def _loss_call(x, q):
    m, k = x.shape
    return pl.pallas_call(
        _loss_body,
        grid=(m // _BL,),
        in_specs=[
            pl.BlockSpec((_BL, k), lambda i: (i, 0)),
            pl.BlockSpec((_BL, k), lambda i: (i, 0)),
        ],
        out_specs=pl.BlockSpec((1, 1), lambda i: (0, 0),
                               memory_space=pltpu.SMEM),
        out_shape=jax.ShapeDtypeStruct((1, 1), jnp.float32),
        compiler_params=pltpu.CompilerParams(
            dimension_semantics=("arbitrary",)),
    )(x, q)


def kernel(x, weight):
    m, k = x.shape
    n = weight.shape[0]
    flat_x = _l2_normalize(x, axis=1)
    w_norm = _l2_normalize(weight, axis=1)
    sx = jnp.sum(flat_x ** 2, axis=1, keepdims=True)
    sw = jnp.sum(w_norm ** 2, axis=1)[None, :]

    idx = _argmin_call(flat_x, w_norm, sx, sw).reshape(-1)
    q = _make_gather(n, k, m)(weight, idx)
    loss = _loss_call(x, q)[0, 0]
    return q, loss, idx
